# baseline (speedup 1.0000x reference)
import functools

import jax
import jax.numpy as jnp
from jax import lax
from jax.experimental import pallas as pl
from jax.experimental.pallas import tpu as pltpu
from jax.experimental.pallas import tpu_sc as plsc

NC = 2
NS = 16
NW = NC * NS
L = 16

_params = pltpu.CompilerParams(needs_layout_passes=False,
                               use_tc_tiling_on_sc=False)


def _mesh():
    return plsc.VectorSubcoreMesh(core_axis_name="c", subcore_axis_name="s",
                                  num_cores=NC, num_subcores=NS)


@functools.lru_cache(maxsize=None)
def _build_repack_kernel(V):
    VB = 2000
    NCHUNK = V // VB

    @functools.partial(
        pl.kernel,
        out_type=[
            jax.ShapeDtypeStruct((V, 8), jnp.float32),
            jax.ShapeDtypeStruct((NW, 6 * L), jnp.float32),
        ],
        mesh=_mesh(),
        compiler_params=_params,
        scratch_types=[
            pltpu.VMEM((VB,), jnp.float32),
            pltpu.VMEM((VB,), jnp.float32),
            pltpu.VMEM((VB,), jnp.float32),
            pltpu.VMEM((VB, 8), jnp.float32),
            pltpu.VMEM((6 * L,), jnp.float32),
        ],
    )
    def repack(vT, packed_out, bpart_out, xb, yb, zb, pb, bacc):
        vx, vy, vz = vT.at[0], vT.at[1], vT.at[2]
        wid = lax.axis_index("s") * NC + lax.axis_index("c")
        lanes = lax.iota(jnp.int32, L)

        inf = jnp.float32(jnp.inf)
        acc0 = tuple(jnp.full((L,), inf, jnp.float32) for _ in range(3)) + \
               tuple(jnp.full((L,), -inf, jnp.float32) for _ in range(3))

        def chunk_body(k, acc):
            chunk = wid + k * NW
            base = chunk * VB
            pltpu.sync_copy(vx.at[pl.ds(base, VB)], xb)
            pltpu.sync_copy(vy.at[pl.ds(base, VB)], yb)
            pltpu.sync_copy(vz.at[pl.ds(base, VB)], zb)

            zero = jnp.full((L,), 0, jnp.int32)

            def step(t, acc):
                o = t * L
                x = xb[pl.ds(o, L)]
                y = yb[pl.ds(o, L)]
                z = zb[pl.ds(o, L)]
                i = o + lanes
                plsc.store_scatter(pb, [i, zero], x)
                plsc.store_scatter(pb, [i, zero + 1], y)
                plsc.store_scatter(pb, [i, zero + 2], z)
                mn0, mn1, mn2, mx0, mx1, mx2 = acc
                return (jnp.minimum(mn0, x), jnp.minimum(mn1, y),
                        jnp.minimum(mn2, z), jnp.maximum(mx0, x),
                        jnp.maximum(mx1, y), jnp.maximum(mx2, z))

            acc = lax.fori_loop(0, VB // L, step, acc)
            pltpu.sync_copy(pb, packed_out.at[pl.ds(chunk * VB, VB), :])
            return acc

        nch = (NCHUNK - 1 - wid) // NW + 1
        acc = lax.fori_loop(0, nch, chunk_body, acc0)
        for i in range(6):
            bacc[pl.ds(i * L, L)] = acc[i]
        pltpu.sync_copy(bacc, bpart_out.at[wid])

    return repack


@functools.lru_cache(maxsize=None)
def _build_main_kernel(V, F):
    C = 640
    NCHUNK = F // C
    G = C // L
    R = (3 * C) // 128

    @functools.partial(
        pl.kernel,
        out_type=[
            jax.ShapeDtypeStruct((F // 128, 4, 128), jnp.float32),
            jax.ShapeDtypeStruct((F // 128, 4, 128), jnp.float32),
            jax.ShapeDtypeStruct((F,), jnp.float32),
        ],
        mesh=_mesh(),
        compiler_params=_params,
        scratch_types=[
            [pltpu.VMEM((R, 128), jnp.int32),
             pltpu.VMEM((3 * C, 8), jnp.float32),
             pltpu.VMEM((C // 128, 4, 128), jnp.float32),
             pltpu.VMEM((C // 128, 4, 128), jnp.float32),
             pltpu.VMEM((C,), jnp.float32),
             [pltpu.SemaphoreType.DMA for _ in range(C // 128)]],
            [pltpu.VMEM((R, 128), jnp.int32),
             pltpu.VMEM((3 * C, 8), jnp.float32),
             pltpu.VMEM((C // 128, 4, 128), jnp.float32),
             pltpu.VMEM((C // 128, 4, 128), jnp.float32),
             pltpu.VMEM((C,), jnp.float32),
             [pltpu.SemaphoreType.DMA for _ in range(C // 128)]],
        ],
    )
    def main(packed, fT3, cent_out, norm_out, area_out, set0, set1):
        f0, f1, f2 = fT3.at[0], fT3.at[1], fT3.at[2]
        wid = lax.axis_index("s") * NC + lax.axis_index("c")
        lanes = lax.iota(jnp.int32, L)
        TPC = C // 128

        def load_and_fire(chunk, bufs):
            qidx_v, rows_v, _, _, _, sems = bufs
            row0 = chunk * TPC
            pltpu.sync_copy(f0.at[pl.ds(row0, TPC), :],
                            qidx_v.at[pl.ds(0, TPC), :])
            pltpu.sync_copy(f1.at[pl.ds(row0, TPC), :],
                            qidx_v.at[pl.ds(TPC, TPC), :])
            pltpu.sync_copy(f2.at[pl.ds(row0, TPC), :],
                            qidx_v.at[pl.ds(2 * TPC, TPC), :])
            for t in range(TPC):
                for k in range(3):
                    s = k * TPC + t
                    pltpu.async_copy(packed.at[qidx_v.at[s]],
                                     rows_v.at[pl.ds(s * 128, 128), :],
                                     sems[t])

        def drain_compute_store(chunk, bufs):
            qidx_v, rows_v, cb, nb, arb, sems = bufs
            zero = jnp.full((L,), 0, jnp.int32)

            def group(g, _):
                o = g * L
                f = o + lanes
                tri = []
                for k in range(3):
                    row = f + k * C
                    for c in range(3):
                        tri.append(plsc.load_gather(rows_v, [row, zero + c]))
                v0x, v0y, v0z, v1x, v1y, v1z, v2x, v2y, v2z = tri

                tb = g >> 3
                pos = (g & 7) * L
                third = jnp.float32(1.0 / 3.0)
                cb[tb, 0, pl.ds(pos, L)] = (v0x + v1x + v2x) * third
                cb[tb, 1, pl.ds(pos, L)] = (v0y + v1y + v2y) * third
                cb[tb, 2, pl.ds(pos, L)] = (v0z + v1z + v2z) * third

                e1x = v1x - v0x
                e1y = v1y - v0y
                e1z = v1z - v0z
                e2x = v2x - v1x
                e2y = v2y - v1y
                e2z = v2z - v1z
                cx = e1y * e2z - e1z * e2y
                cy = e1z * e2x - e1x * e2z
                cz = e1x * e2y - e1y * e2x
                s = cx * cx + cy * cy + cz * cz
                bits = plsc.bitcast(s, jnp.int32)
                y = plsc.bitcast(jnp.int32(0x5F3759DF) - (bits >> 1),
                                 jnp.float32)
                half_s = s * 0.5
                for _ in range(2):
                    y = y * (1.5 - half_s * y * y)
                nb[tb, 0, pl.ds(pos, L)] = cx * y
                nb[tb, 1, pl.ds(pos, L)] = cy * y
                nb[tb, 2, pl.ds(pos, L)] = cz * y
                arb[pl.ds(o, L)] = (s * y) * 0.5
                return 0

            for t in range(TPC):
                for k in range(3):
                    s = k * TPC + t
                    pltpu.make_async_copy(
                        packed.at[qidx_v.at[s]],
                        rows_v.at[pl.ds(s * 128, 128), :], sems[t]).wait()
                lax.fori_loop(t * 8, (t + 1) * 8, group, 0)
            fbase = chunk * C
            tile0 = chunk * (C // 128)
            pltpu.sync_copy(cb, cent_out.at[pl.ds(tile0, C // 128), :, :])
            pltpu.sync_copy(nb, norm_out.at[pl.ds(tile0, C // 128), :, :])
            pltpu.sync_copy(arb, area_out.at[pl.ds(fbase, C)])

        def guarded(stage, chunk, bufs):
            @pl.when(chunk < NCHUNK)
            def _():
                stage(chunk, bufs)

        guarded(load_and_fire, wid, set0)

        def pair(m, _):
            c0 = wid + (2 * m) * NW
            c1 = c0 + NW
            c2 = c1 + NW
            guarded(load_and_fire, c1, set1)
            guarded(drain_compute_store, c0, set0)
            guarded(load_and_fire, c2, set0)
            guarded(drain_compute_store, c1, set1)
            return 0

        npair = (NCHUNK + 2 * NW - 1) // (2 * NW)
        lax.fori_loop(0, npair, pair, 0)

    return main


def kernel(vertices, faces):
    V = vertices.shape[0]
    F = faces.shape[0]
    vT = vertices.T
    fT3 = faces.T.reshape(3, F // 128, 128)
    packed, bpart = _build_repack_kernel(V)(vT)
    cent, norm, area = _build_main_kernel(V, F)(packed, fT3)
    face_centers = cent.transpose(0, 2, 1).reshape(F, 4)[:, :3]
    face_normals = norm.transpose(0, 2, 1).reshape(F, 4)[:, :3]
    bpart = bpart.reshape(NW, 6, L)
    mins = jnp.min(bpart[:, 0:3, :], axis=(0, 2))
    maxs = jnp.max(bpart[:, 3:6, :], axis=(0, 2))
    bounds = jnp.stack([mins, maxs], axis=-1)
    return face_centers, face_normals, area, bounds


# speedup vs baseline: 1.5830x; 1.5830x over previous
import functools

import jax
import jax.numpy as jnp
from jax import lax
from jax.experimental import pallas as pl
from jax.experimental.pallas import tpu as pltpu
from jax.experimental.pallas import tpu_sc as plsc

NC = 2
NS = 16
NW = NC * NS
L = 16

_params = pltpu.CompilerParams(needs_layout_passes=False,
                               use_tc_tiling_on_sc=False)


def _mesh():
    return plsc.VectorSubcoreMesh(core_axis_name="c", subcore_axis_name="s",
                                  num_cores=NC, num_subcores=NS)


@functools.lru_cache(maxsize=None)
def _build_repack_kernel(V):
    VB = 2000
    NCHUNK = V // VB

    @functools.partial(
        pl.kernel,
        out_type=[
            jax.ShapeDtypeStruct((V, 8), jnp.float32),
            jax.ShapeDtypeStruct((NW, 6 * L), jnp.float32),
        ],
        mesh=_mesh(),
        compiler_params=_params,
        scratch_types=[
            pltpu.VMEM((VB,), jnp.float32),
            pltpu.VMEM((VB,), jnp.float32),
            pltpu.VMEM((VB,), jnp.float32),
            pltpu.VMEM((VB, 8), jnp.float32),
            pltpu.VMEM((6 * L,), jnp.float32),
        ],
    )
    def repack(vx, vy, vz, packed_out, bpart_out, xb, yb, zb, pb, bacc):
        wid = lax.axis_index("s") * NC + lax.axis_index("c")
        lanes = lax.iota(jnp.int32, L)

        inf = jnp.float32(jnp.inf)
        acc0 = tuple(jnp.full((L,), inf, jnp.float32) for _ in range(3)) + \
               tuple(jnp.full((L,), -inf, jnp.float32) for _ in range(3))

        def chunk_body(k, acc):
            chunk = wid + k * NW
            base = chunk * VB
            pltpu.sync_copy(vx.at[pl.ds(base, VB)], xb)
            pltpu.sync_copy(vy.at[pl.ds(base, VB)], yb)
            pltpu.sync_copy(vz.at[pl.ds(base, VB)], zb)

            zero = jnp.full((L,), 0, jnp.int32)

            def step(t, acc):
                o = t * L
                x = xb[pl.ds(o, L)]
                y = yb[pl.ds(o, L)]
                z = zb[pl.ds(o, L)]
                i = o + lanes
                plsc.store_scatter(pb, [i, zero], x)
                plsc.store_scatter(pb, [i, zero + 1], y)
                plsc.store_scatter(pb, [i, zero + 2], z)
                mn0, mn1, mn2, mx0, mx1, mx2 = acc
                return (jnp.minimum(mn0, x), jnp.minimum(mn1, y),
                        jnp.minimum(mn2, z), jnp.maximum(mx0, x),
                        jnp.maximum(mx1, y), jnp.maximum(mx2, z))

            acc = lax.fori_loop(0, VB // L, step, acc)
            pltpu.sync_copy(pb, packed_out.at[pl.ds(chunk * VB, VB), :])
            return acc

        nch = (NCHUNK - 1 - wid) // NW + 1
        acc = lax.fori_loop(0, nch, chunk_body, acc0)
        for i in range(6):
            bacc[pl.ds(i * L, L)] = acc[i]
        pltpu.sync_copy(bacc, bpart_out.at[wid])

    return repack


@functools.lru_cache(maxsize=None)
def _build_main_kernel(V, F):
    C = 640
    NCHUNK = F // C
    G = C // L
    R = (3 * C) // 128

    @functools.partial(
        pl.kernel,
        out_type=[
            jax.ShapeDtypeStruct((F // 128, 4, 128), jnp.float32),
            jax.ShapeDtypeStruct((F // 128, 4, 128), jnp.float32),
            jax.ShapeDtypeStruct((F,), jnp.float32),
        ],
        mesh=_mesh(),
        compiler_params=_params,
        scratch_types=[
            [pltpu.VMEM((R, 128), jnp.int32),
             pltpu.VMEM((3 * C, 8), jnp.float32),
             pltpu.VMEM((C // 128, 4, 128), jnp.float32),
             pltpu.VMEM((C // 128, 4, 128), jnp.float32),
             pltpu.VMEM((C,), jnp.float32),
             [pltpu.SemaphoreType.DMA for _ in range(C // 128)],
             pltpu.SemaphoreType.DMA],
            [pltpu.VMEM((R, 128), jnp.int32),
             pltpu.VMEM((3 * C, 8), jnp.float32),
             pltpu.VMEM((C // 128, 4, 128), jnp.float32),
             pltpu.VMEM((C // 128, 4, 128), jnp.float32),
             pltpu.VMEM((C,), jnp.float32),
             [pltpu.SemaphoreType.DMA for _ in range(C // 128)],
             pltpu.SemaphoreType.DMA],
        ],
    )
    def main(packed, f0, f1, f2, cent_out, norm_out, area_out, set0, set1):
        wid = lax.axis_index("s") * NC + lax.axis_index("c")
        lanes = lax.iota(jnp.int32, L)
        TPC = C // 128

        def load_and_fire(chunk, bufs):
            qidx_v, rows_v, _, _, _, sems, _ = bufs
            row0 = chunk * TPC
            pltpu.sync_copy(f0.at[pl.ds(row0, TPC), :],
                            qidx_v.at[pl.ds(0, TPC), :])
            pltpu.sync_copy(f1.at[pl.ds(row0, TPC), :],
                            qidx_v.at[pl.ds(TPC, TPC), :])
            pltpu.sync_copy(f2.at[pl.ds(row0, TPC), :],
                            qidx_v.at[pl.ds(2 * TPC, TPC), :])
            for t in range(TPC):
                for k in range(3):
                    s = k * TPC + t
                    pltpu.async_copy(packed.at[qidx_v.at[s]],
                                     rows_v.at[pl.ds(s * 128, 128), :],
                                     sems[t])

        def out_copies(chunk, bufs):
            _, _, cb, nb, arb, _, osem = bufs
            fbase = chunk * C
            tile0 = chunk * TPC
            return (
                pltpu.make_async_copy(
                    cb, cent_out.at[pl.ds(tile0, TPC), :, :], osem),
                pltpu.make_async_copy(
                    nb, norm_out.at[pl.ds(tile0, TPC), :, :], osem),
                pltpu.make_async_copy(arb, area_out.at[pl.ds(fbase, C)],
                                      osem),
            )

        def drain_compute_store(chunk, bufs):
            qidx_v, rows_v, cb, nb, arb, sems, osem = bufs

            @pl.when(chunk >= wid + 2 * NW)
            def _():
                for d in out_copies(chunk - 2 * NW, bufs):
                    d.wait()

            zero = jnp.full((L,), 0, jnp.int32)

            def group(g, _):
                o = g * L
                f = o + lanes
                tri = []
                for k in range(3):
                    row = f + k * C
                    for c in range(3):
                        tri.append(plsc.load_gather(rows_v, [row, zero + c]))
                v0x, v0y, v0z, v1x, v1y, v1z, v2x, v2y, v2z = tri

                tb = g >> 3
                pos = (g & 7) * L
                third = jnp.float32(1.0 / 3.0)
                cb[tb, 0, pl.ds(pos, L)] = (v0x + v1x + v2x) * third
                cb[tb, 1, pl.ds(pos, L)] = (v0y + v1y + v2y) * third
                cb[tb, 2, pl.ds(pos, L)] = (v0z + v1z + v2z) * third

                e1x = v1x - v0x
                e1y = v1y - v0y
                e1z = v1z - v0z
                e2x = v2x - v1x
                e2y = v2y - v1y
                e2z = v2z - v1z
                cx = e1y * e2z - e1z * e2y
                cy = e1z * e2x - e1x * e2z
                cz = e1x * e2y - e1y * e2x
                s = cx * cx + cy * cy + cz * cz
                bits = plsc.bitcast(s, jnp.int32)
                y = plsc.bitcast(jnp.int32(0x5F3759DF) - (bits >> 1),
                                 jnp.float32)
                half_s = s * 0.5
                for _ in range(2):
                    y = y * (1.5 - half_s * y * y)
                nb[tb, 0, pl.ds(pos, L)] = cx * y
                nb[tb, 1, pl.ds(pos, L)] = cy * y
                nb[tb, 2, pl.ds(pos, L)] = cz * y
                arb[pl.ds(o, L)] = (s * y) * 0.5
                return 0

            for t in range(TPC):
                for k in range(3):
                    s = k * TPC + t
                    pltpu.make_async_copy(
                        packed.at[qidx_v.at[s]],
                        rows_v.at[pl.ds(s * 128, 128), :], sems[t]).wait()
                lax.fori_loop(t * 8, (t + 1) * 8, group, 0)
            for d in out_copies(chunk, bufs):
                d.start()

        def guarded(stage, chunk, bufs):
            @pl.when(chunk < NCHUNK)
            def _():
                stage(chunk, bufs)

        guarded(load_and_fire, wid, set0)

        def pair(m, _):
            c0 = wid + (2 * m) * NW
            c1 = c0 + NW
            c2 = c1 + NW
            guarded(load_and_fire, c1, set1)
            guarded(drain_compute_store, c0, set0)
            guarded(load_and_fire, c2, set0)
            guarded(drain_compute_store, c1, set1)
            return 0

        npair = (NCHUNK + 2 * NW - 1) // (2 * NW)
        lax.fori_loop(0, npair, pair, 0)
        for st in (set0, set1):
            for d in out_copies(wid, st):
                d.wait()

    return main


def kernel(vertices, faces):
    V = vertices.shape[0]
    F = faces.shape[0]
    vx, vy, vz = vertices[:, 0], vertices[:, 1], vertices[:, 2]
    f0 = faces[:, 0].reshape(F // 128, 128)
    f1 = faces[:, 1].reshape(F // 128, 128)
    f2 = faces[:, 2].reshape(F // 128, 128)
    packed, bpart = _build_repack_kernel(V)(vx, vy, vz)
    cent, norm, area = _build_main_kernel(V, F)(packed, f0, f1, f2)
    face_centers = cent.transpose(0, 2, 1).reshape(F, 4)[:, :3]
    face_normals = norm.transpose(0, 2, 1).reshape(F, 4)[:, :3]
    bpart = bpart.reshape(NW, 6, L)
    mins = jnp.min(bpart[:, 0:3, :], axis=(0, 2))
    maxs = jnp.max(bpart[:, 3:6, :], axis=(0, 2))
    bounds = jnp.stack([mins, maxs], axis=-1)
    return face_centers, face_normals, area, bounds
